# Initial kernel scaffold; baseline (speedup 1.0000x reference)
import jax
import jax.numpy as jnp
from jax.experimental import pallas as pl


def kernel(edge_index, adj_values, input_feature, weight, bias):
    raise NotImplementedError("write your pallas kernel here")


# speedup vs baseline: 4.5025x; 4.5025x over previous
import functools

import jax
import jax.numpy as jnp
from jax import lax
from jax.experimental import pallas as pl
from jax.experimental.pallas import tpu as pltpu
from jax.experimental.pallas import tpu_sc as plsc

N_CORES = 2
N_TILES = 16
LANES = 16
CHUNK = 128
HALF = 64


def _mm_body(x_ref, w_ref, o_ref):
    o_ref[...] = jnp.dot(x_ref[...], w_ref[0],
                         preferred_element_type=jnp.float32)


def _support_colsplit(x, w):
    n = x.shape[0]
    rb = 1000
    nrb = n // rb
    ws = w.reshape(w.shape[0], N_CORES, HALF).transpose(1, 0, 2)
    return pl.pallas_call(
        _mm_body,
        grid=(N_CORES, nrb),
        in_specs=[
            pl.BlockSpec((rb, x.shape[1]), lambda h, i: (i, 0)),
            pl.BlockSpec((1, x.shape[1], HALF), lambda h, i: (h, 0, 0)),
        ],
        out_specs=pl.BlockSpec((rb, HALF), lambda h, i: (h * nrb + i, 0)),
        out_shape=jax.ShapeDtypeStruct((N_CORES * n, HALF), jnp.float32),
    )(x, ws)


def _make_agg(n_nodes, nchunk):
    rows_per_tile = n_nodes // N_TILES
    epil = CHUNK - 3
    n_init = rows_per_tile // epil
    mesh = plsc.VectorSubcoreMesh(core_axis_name="c", subcore_axis_name="s")

    @functools.partial(
        pl.kernel,
        out_type=jax.ShapeDtypeStruct((n_nodes, 2 * HALF), jnp.float32),
        mesh=mesh,
        compiler_params=pltpu.CompilerParams(
            use_tc_tiling_on_sc=False, needs_layout_passes=False),
        scratch_types=[
            pltpu.VMEM((nchunk, CHUNK), jnp.int32),
            pltpu.VMEM((nchunk, CHUNK), jnp.int32),
            pltpu.VMEM((nchunk, CHUNK), jnp.float32),
            pltpu.VMEM((CHUNK, HALF), jnp.float32),
            pltpu.VMEM((2 * HALF,), jnp.float32),
            pltpu.VMEM_SHARED((n_nodes, HALF), jnp.float32),
        ],
    )
    def agg(support_ref, src_ref, dst_ref, val_ref, bias_ref, out_ref,
            src_v, dst_v, val_v, rows, bias_v, acc):
        c = lax.axis_index("c")
        sid = lax.axis_index("s")

        pltpu.sync_copy(src_ref.at[sid], src_v)
        pltpu.sync_copy(dst_ref.at[sid], dst_v)
        pltpu.sync_copy(val_ref.at[sid], val_v)
        pltpu.sync_copy(bias_ref, bias_v)

        coff = c * n_nodes
        def add_off(i, carry):
            for k in range(CHUNK // LANES):
                sl = pl.ds(k * LANES, LANES)
                src_v[i, sl] = src_v[i, sl] + coff
            return carry
        lax.fori_loop(0, nchunk, add_off, None)

        bvs = [bias_v[pl.ds(c * HALF + k * LANES, LANES)]
               for k in range(HALF // LANES)]
        def bias_row(r, carry):
            for k in range(HALF // LANES):
                rows[r, pl.ds(k * LANES, LANES)] = bvs[k]
            return carry
        lax.fori_loop(0, epil, bias_row, None)
        base = sid * rows_per_tile
        for k in range(n_init):
            pltpu.sync_copy(rows.at[pl.ds(0, epil)],
                            acc.at[pl.ds(base + k * epil, epil)])
        plsc.subcore_barrier()

        def chunk_body(j, carry):
            pltpu.sync_copy(support_ref.at[src_v.at[j]], rows)

            def scale_edge(e, carry2):
                vb = plsc.load_gather(
                    val_v, [jnp.full((LANES,), j, jnp.int32),
                            jnp.full((LANES,), e, jnp.int32)])
                for k in range(HALF // LANES):
                    sl = pl.ds(k * LANES, LANES)
                    rows[e, sl] = rows[e, sl] * vb
                return carry2
            lax.fori_loop(0, CHUNK, scale_edge, None)

            pltpu.sync_copy(rows, acc.at[dst_v.at[j]], add=True)
            return carry
        lax.fori_loop(0, nchunk, chunk_body, None)
        plsc.subcore_barrier()

        pltpu.sync_copy(
            acc.at[pl.ds(base, rows_per_tile)],
            out_ref.at[pl.ds(base, rows_per_tile), pl.ds(c * HALF, HALF)])

    return agg


def kernel(edge_index, adj_values, input_feature, weight, bias):
    n_nodes = input_feature.shape[0]
    n_edges = adj_values.shape[0]
    src = edge_index[0].astype(jnp.int32)
    dst = edge_index[1].astype(jnp.int32)

    per_tile = -(-n_edges // (N_TILES * CHUNK)) * CHUNK
    e_pad = N_TILES * per_tile
    pad = e_pad - n_edges
    nchunk = per_tile // CHUNK
    src_p = jnp.pad(src, (0, pad)).reshape(N_TILES, nchunk, CHUNK)
    dst_p = jnp.pad(dst, (0, pad)).reshape(N_TILES, nchunk, CHUNK)
    val_p = jnp.pad(adj_values, (0, pad)).reshape(N_TILES, nchunk, CHUNK)

    support = _support_colsplit(input_feature, weight)
    agg = _make_agg(n_nodes, nchunk)
    return agg(support, src_p, dst_p, val_p, bias)
